# baseline (speedup 1.0000x reference)
import jax
import jax.numpy as jnp
from jax import lax
from jax.experimental import pallas as pl
from jax.experimental.pallas import tpu as pltpu
from jax.experimental.pallas import tpu_sc as plsc

_VOCAB = 100000
_HID = 128
_NV = _HID // 16
_SEQ = 200
_BATCH = 1024
_EPS = 1e-12
_NW = 32
_TOK = _BATCH * _SEQ
_PER_W = _TOK // _NW
_CHUNK = 128
_NCHUNK = _PER_W // _CHUNK
_UNROLL = 16 if False else 16


def _rsqrt16(v):
    i = plsc.bitcast(v, jnp.int32)
    i = jnp.int32(0x5F3759DF) - (i >> 1)
    y = plsc.bitcast(i, jnp.float32)
    half = v * jnp.float32(0.5)
    for _ in range(3):
        y = y * (jnp.float32(1.5) - half * y * y)
    return y


def _tree_sum(vs):
    while len(vs) > 1:
        vs = [a + b for a, b in zip(vs[::2], vs[1::2])]
    return vs[0]


def _tec_body(ids_hbm, tt_hbm, word_hbm, pos_hbm, type_hbm, gam_hbm, bet_hbm,
              out_hbm, pt_v, ty_v, wrows_v, idsw_v, ttw_v, gam_v, bet_v,
              stats_v, prow_v, sums_v, gsem, osem):
    wid = lax.axis_index("s") * 2 + lax.axis_index("c")

    pltpu.sync_copy(ids_hbm.at[pl.ds(wid * _PER_W, _PER_W)], idsw_v)
    pltpu.sync_copy(tt_hbm.at[pl.ds(wid * _PER_W, _PER_W)], ttw_v)
    pltpu.sync_copy(pos_hbm.at[pl.ds(0, _SEQ)], pt_v.at[pl.ds(0, _SEQ)])
    pltpu.sync_copy(pos_hbm.at[pl.ds(0, _SEQ)], pt_v.at[pl.ds(_SEQ, _SEQ)])
    pltpu.sync_copy(type_hbm, ty_v)
    pltpu.sync_copy(gam_hbm, gam_v)
    pltpu.sync_copy(bet_hbm, bet_v)

    def add_type(r, _):
        t = r // _SEQ
        for j in range(_NV):
            sl = pl.ds(j * 16, 16)
            pt_v[r, sl] = pt_v[r, sl] + ty_v[t, sl]
        return 0
    lax.fori_loop(0, 2 * _SEQ, add_type, 0)

    gvs = [gam_v[pl.ds(j * 16, 16)] for j in range(_NV)]
    bvs = [bet_v[pl.ds(j * 16, 16)] for j in range(_NV)]

    def idx_slice(c):
        return idsw_v.at[pl.ds(pl.multiple_of(c * _CHUNK, _CHUNK), _CHUNK)]

    pltpu.async_copy(word_hbm.at[idx_slice(0)], wrows_v.at[0], gsem.at[0])

    def chunk_body(c, _):
        p = c & 1
        np_ = 1 - p

        @pl.when(c + 1 < _NCHUNK)
        def _start_next():
            @pl.when(c >= 1)
            def _wait_out():
                pltpu.make_async_copy(
                    wrows_v.at[np_], out_hbm.at[pl.ds(0, _CHUNK)], osem.at[np_]
                ).wait()
            pltpu.async_copy(
                word_hbm.at[idx_slice(c + 1)], wrows_v.at[np_], gsem.at[np_])

        pltpu.make_async_copy(
            word_hbm.at[idx_slice(c)], wrows_v.at[p], gsem.at[p]).wait()

        base = wid * _PER_W + c * _CHUNK
        iota = lax.iota(jnp.int32, 16)

        m15 = lax.eq(iota, jnp.int32(15))
        cols = [jnp.int32(j * 16) + iota for j in range(_NV)]

        def grp_body(g, _unused):
            tvec = ttw_v[pl.ds(c * _CHUNK + g * 16, 16)]
            lvec = lax.rem(base + g * 16 + iota, jnp.int32(_SEQ))
            ptrow = tvec * _SEQ + lvec
            prow_v[pl.ds(0, 16)] = ptrow

            @plsc.parallel_loop(0, 16, step=1, unroll=8)
            def rpass(u):
                i = g * 16 + u
                fu = jnp.full((16,), u, jnp.int32)
                pr16 = plsc.load_gather(prow_v, [fu])
                rsum = jnp.zeros((16,), jnp.float32)
                rsq = jnp.zeros((16,), jnp.float32)
                for j in range(_NV):
                    x = wrows_v[p, i, pl.ds(j * 16, 16)] \
                        + plsc.load_gather(pt_v, [pr16, cols[j]])
                    wrows_v[p, i, pl.ds(j * 16, 16)] = x
                    rsum = rsum + x
                    rsq = rsq + x * x
                cs = plsc.cumsum(rsum)
                cq = plsc.cumsum(rsq)
                plsc.store_scatter(sums_v, [fu], cs, mask=m15)
                plsc.store_scatter(sums_v, [fu + 16], cq, mask=m15)

            s1 = sums_v[pl.ds(0, 16)]
            s2 = sums_v[pl.ds(16, 16)]
            m_v = s1 * jnp.float32(1.0 / _HID)
            var_v = s2 * jnp.float32(1.0 / _HID) - m_v * m_v
            c1 = _rsqrt16(var_v + jnp.float32(_EPS))
            c2 = m_v * c1
            stats_v[pl.ds(0, 16)] = c1
            stats_v[pl.ds(16, 16)] = c2

            @plsc.parallel_loop(0, 16, step=1, unroll=4)
            def norm(u):
                i = g * 16 + u
                fu = jnp.full((16,), u, jnp.int32)
                c1u = plsc.load_gather(stats_v, [fu])
                c2u = plsc.load_gather(stats_v, [fu + 16])
                for j in range(_NV):
                    x = wrows_v[p, i, pl.ds(j * 16, 16)]
                    wrows_v[p, i, pl.ds(j * 16, 16)] = \
                        (x * c1u - c2u) * gvs[j] + bvs[j]
            return 0

        lax.fori_loop(0, _CHUNK // 16, grp_body, 0)

        pltpu.async_copy(wrows_v.at[p], out_hbm.at[pl.ds(base, _CHUNK)],
                         osem.at[p])
        return 0

    lax.fori_loop(0, _NCHUNK, chunk_body, 0)

    pltpu.make_async_copy(
        wrows_v.at[0], out_hbm.at[pl.ds(0, _CHUNK)], osem.at[0]).wait()
    pltpu.make_async_copy(
        wrows_v.at[1], out_hbm.at[pl.ds(0, _CHUNK)], osem.at[1]).wait()


def kernel(input_ids, token_type_ids, word_emb, pos_emb, type_emb, ln_gamma, ln_beta):
    ids = input_ids.reshape(-1).astype(jnp.int32)
    tt = token_type_ids.reshape(-1).astype(jnp.int32)
    mesh = plsc.VectorSubcoreMesh(core_axis_name="c", subcore_axis_name="s")
    k = pl.kernel(
        _tec_body,
        out_type=jax.ShapeDtypeStruct((_TOK, _HID), jnp.float32),
        mesh=mesh,
        scratch_types=[
            pltpu.VMEM((2 * _SEQ, _HID), jnp.float32),
            pltpu.VMEM((2, _HID), jnp.float32),
            pltpu.VMEM((2, _CHUNK, _HID), jnp.float32),
            pltpu.VMEM((_PER_W,), jnp.int32),
            pltpu.VMEM((_PER_W,), jnp.int32),
            pltpu.VMEM((_HID,), jnp.float32),
            pltpu.VMEM((_HID,), jnp.float32),
            pltpu.VMEM((32,), jnp.float32),
            pltpu.VMEM((16,), jnp.int32),
            pltpu.VMEM((32,), jnp.float32),
            pltpu.SemaphoreType.DMA((2,)),
            pltpu.SemaphoreType.DMA((2,)),
        ],
        compiler_params=pltpu.CompilerParams(needs_layout_passes=False),
    )
    out = k(ids, tt, word_emb, pos_emb, type_emb, ln_gamma, ln_beta)
    return out.reshape(_BATCH, _SEQ, _HID)


# speedup vs baseline: 1.2666x; 1.2666x over previous
import jax
import jax.numpy as jnp
from jax import lax
from jax.experimental import pallas as pl
from jax.experimental.pallas import tpu as pltpu
from jax.experimental.pallas import tpu_sc as plsc

_VOCAB = 100000
_HID = 128
_NV = _HID // 16
_SEQ = 200
_BATCH = 1024
_EPS = 1e-12
_NW = 32
_TOK = _BATCH * _SEQ
_PER_W = _TOK // _NW
_CHUNK = 128
_NCHUNK = _PER_W // _CHUNK
_UNROLL = 16 if False else 16


def _rsqrt16(v):
    i = plsc.bitcast(v, jnp.int32)
    i = jnp.int32(0x5F3759DF) - (i >> 1)
    y = plsc.bitcast(i, jnp.float32)
    half = v * jnp.float32(0.5)
    for _ in range(3):
        y = y * (jnp.float32(1.5) - half * y * y)
    return y


def _tree_sum(vs):
    while len(vs) > 1:
        vs = [a + b for a, b in zip(vs[::2], vs[1::2])]
    return vs[0]


def _tec_body(ids_hbm, tt_hbm, word_hbm, pos_hbm, type_hbm, gam_hbm, bet_hbm,
              out_hbm, pt_v, ty_v, wrows_v, idsw_v, ttw_v, gam_v, bet_v,
              stats_v, prow_v, sums_v, gsem, osem):
    wid = lax.axis_index("s") * 2 + lax.axis_index("c")

    pltpu.sync_copy(ids_hbm.at[pl.ds(wid * _PER_W, _PER_W)], idsw_v)
    pltpu.sync_copy(tt_hbm.at[pl.ds(wid * _PER_W, _PER_W)], ttw_v)
    pltpu.sync_copy(pos_hbm.at[pl.ds(0, _SEQ)], pt_v.at[pl.ds(0, _SEQ)])
    pltpu.sync_copy(pos_hbm.at[pl.ds(0, _SEQ)], pt_v.at[pl.ds(_SEQ, _SEQ)])
    pltpu.sync_copy(type_hbm, ty_v)
    pltpu.sync_copy(gam_hbm, gam_v)
    pltpu.sync_copy(bet_hbm, bet_v)

    def add_type(r, _):
        t = r // _SEQ
        for j in range(_NV):
            sl = pl.ds(j * 16, 16)
            pt_v[r, sl] = pt_v[r, sl] + ty_v[t, sl]
        return 0
    lax.fori_loop(0, 2 * _SEQ, add_type, 0)

    gvs = [gam_v[pl.ds(j * 16, 16)] for j in range(_NV)]
    bvs = [bet_v[pl.ds(j * 16, 16)] for j in range(_NV)]

    def idx_slice(c):
        return idsw_v.at[pl.ds(pl.multiple_of(c * _CHUNK, _CHUNK), _CHUNK)]

    pltpu.async_copy(word_hbm.at[idx_slice(0)], wrows_v.at[0], gsem.at[0])

    def chunk_body(c, _):
        p = c & 1
        np_ = 1 - p

        @pl.when(c + 1 < _NCHUNK)
        def _start_next():
            @pl.when(c >= 1)
            def _wait_out():
                pltpu.make_async_copy(
                    wrows_v.at[np_], out_hbm.at[pl.ds(0, _CHUNK)], osem.at[np_]
                ).wait()
            pltpu.async_copy(
                word_hbm.at[idx_slice(c + 1)], wrows_v.at[np_], gsem.at[np_])

        pltpu.make_async_copy(
            word_hbm.at[idx_slice(c)], wrows_v.at[p], gsem.at[p]).wait()

        base = wid * _PER_W + c * _CHUNK
        iota = lax.iota(jnp.int32, 16)

        m15 = lax.eq(iota, jnp.int32(15))
        cols = [jnp.int32(j * 16) + iota for j in range(_NV)]

        def fill_prow(g, _unused):
            tvec = ttw_v[pl.ds(c * _CHUNK + g * 16, 16)]
            lvec = lax.rem(base + g * 16 + iota, jnp.int32(_SEQ))
            prow_v[pl.ds(pl.multiple_of(g * 16, 16), 16)] = tvec * _SEQ + lvec
            return 0
        lax.fori_loop(0, _CHUNK // 16, fill_prow, 0)

        @plsc.parallel_loop(0, _CHUNK, step=1, unroll=4)
        def rpass(u):
            fu = jnp.full((16,), u, jnp.int32)
            pr16 = plsc.load_gather(prow_v, [fu])
            rsum = jnp.zeros((16,), jnp.float32)
            rsq = jnp.zeros((16,), jnp.float32)
            for j in range(_NV):
                x = wrows_v[p, u, pl.ds(j * 16, 16)] \
                    + plsc.load_gather(pt_v, [pr16, cols[j]])
                wrows_v[p, u, pl.ds(j * 16, 16)] = x
                rsum = rsum + x
                rsq = rsq + x * x
            cs = plsc.cumsum(rsum)
            cq = plsc.cumsum(rsq)
            plsc.store_scatter(sums_v, [fu], cs, mask=m15)
            plsc.store_scatter(sums_v, [fu + _CHUNK], cq, mask=m15)

        def stats_body(g, _unused):
            sl = pl.ds(pl.multiple_of(g * 16, 16), 16)
            s1 = sums_v[sl]
            s2 = sums_v[pl.ds(pl.multiple_of(_CHUNK + g * 16, 16), 16)]
            m_v = s1 * jnp.float32(1.0 / _HID)
            var_v = s2 * jnp.float32(1.0 / _HID) - m_v * m_v
            c1 = _rsqrt16(var_v + jnp.float32(_EPS))
            stats_v[sl] = c1
            stats_v[pl.ds(pl.multiple_of(_CHUNK + g * 16, 16), 16)] = m_v * c1
            return 0
        lax.fori_loop(0, _CHUNK // 16, stats_body, 0)

        @plsc.parallel_loop(0, _CHUNK, step=1, unroll=4)
        def norm(u):
            fu = jnp.full((16,), u, jnp.int32)
            c1u = plsc.load_gather(stats_v, [fu])
            c2u = plsc.load_gather(stats_v, [fu + _CHUNK])
            for j in range(_NV):
                x = wrows_v[p, u, pl.ds(j * 16, 16)]
                wrows_v[p, u, pl.ds(j * 16, 16)] = \
                    (x * c1u - c2u) * gvs[j] + bvs[j]

        pltpu.async_copy(wrows_v.at[p], out_hbm.at[pl.ds(base, _CHUNK)],
                         osem.at[p])
        return 0

    lax.fori_loop(0, _NCHUNK, chunk_body, 0)

    pltpu.make_async_copy(
        wrows_v.at[0], out_hbm.at[pl.ds(0, _CHUNK)], osem.at[0]).wait()
    pltpu.make_async_copy(
        wrows_v.at[1], out_hbm.at[pl.ds(0, _CHUNK)], osem.at[1]).wait()


def kernel(input_ids, token_type_ids, word_emb, pos_emb, type_emb, ln_gamma, ln_beta):
    ids = input_ids.reshape(-1).astype(jnp.int32)
    tt = token_type_ids.reshape(-1).astype(jnp.int32)
    mesh = plsc.VectorSubcoreMesh(core_axis_name="c", subcore_axis_name="s")
    k = pl.kernel(
        _tec_body,
        out_type=jax.ShapeDtypeStruct((_TOK, _HID), jnp.float32),
        mesh=mesh,
        scratch_types=[
            pltpu.VMEM((2 * _SEQ, _HID), jnp.float32),
            pltpu.VMEM((2, _HID), jnp.float32),
            pltpu.VMEM((2, _CHUNK, _HID), jnp.float32),
            pltpu.VMEM((_PER_W,), jnp.int32),
            pltpu.VMEM((_PER_W,), jnp.int32),
            pltpu.VMEM((_HID,), jnp.float32),
            pltpu.VMEM((_HID,), jnp.float32),
            pltpu.VMEM((2 * _CHUNK,), jnp.float32),
            pltpu.VMEM((_CHUNK,), jnp.int32),
            pltpu.VMEM((2 * _CHUNK,), jnp.float32),
            pltpu.SemaphoreType.DMA((2,)),
            pltpu.SemaphoreType.DMA((2,)),
        ],
        compiler_params=pltpu.CompilerParams(needs_layout_passes=False),
    )
    out = k(ids, tt, word_emb, pos_emb, type_emb, ln_gamma, ln_beta)
    return out.reshape(_BATCH, _SEQ, _HID)
